# Initial kernel scaffold; baseline (speedup 1.0000x reference)
import jax
import jax.numpy as jnp
from jax.experimental import pallas as pl


def kernel(x, edge_index, W, b):
    raise NotImplementedError("write your pallas kernel here")


# speedup vs baseline: 13.2173x; 13.2173x over previous
import functools

import jax
import jax.numpy as jnp
from jax import lax
from jax.experimental import pallas as pl
from jax.experimental.pallas import tpu as pltpu
from jax.experimental.pallas import tpu_sc as plsc

NC = 2
NS = 16
NW = NC * NS
L = 16


def _gather_max(xt, idx_s, idx_d, n_chunks, chunk_nodes, K, C, npw):
    BN = xt.shape[0]
    GK = chunk_nodes * K
    mesh = plsc.VectorSubcoreMesh(core_axis_name="c", subcore_axis_name="s")

    @functools.partial(
        pl.kernel,
        out_type=jax.ShapeDtypeStruct((BN, C), jnp.float32),
        mesh=mesh,
        scratch_types=[
            pltpu.VMEM((n_chunks, GK), jnp.int32),
            pltpu.VMEM((n_chunks, GK), jnp.int32),
            pltpu.VMEM((GK, C), jnp.float32),
            pltpu.VMEM((GK, C), jnp.float32),
            pltpu.VMEM((chunk_nodes, C), jnp.float32),
            pltpu.SemaphoreType.DMA,
            pltpu.SemaphoreType.DMA,
        ],
        compiler_params=pltpu.CompilerParams(use_tc_tiling_on_sc=False),
    )
    def body(xt_hbm, ids_hbm, idd_hbm, out_hbm, ids_v, idd_v, rs_v, rd_v, o_v,
             sem_s, sem_d):
        wid = lax.axis_index("s") * NC + lax.axis_index("c")
        pltpu.sync_copy(ids_hbm.at[wid], ids_v)
        pltpu.sync_copy(idd_hbm.at[wid], idd_v)
        node0 = wid * npw

        def chunk_body(g, carry):
            cs = pltpu.async_copy(xt_hbm.at[ids_v.at[g]], rs_v, sem_s)
            cd = pltpu.async_copy(xt_hbm.at[idd_v.at[g]], rd_v, sem_d)
            cs.wait()
            cd.wait()
            for j in range(chunk_nodes):
                r0 = j * K
                for cb in range(C // L):
                    sl = pl.ds(cb * L, L)
                    acc = rs_v[r0, sl] - rd_v[r0, sl]
                    for k in range(1, K):
                        acc = jnp.maximum(acc, rs_v[r0 + k, sl] - rd_v[r0 + k, sl])
                    o_v[j, sl] = acc
            pltpu.sync_copy(o_v, out_hbm.at[pl.ds(node0 + g * chunk_nodes,
                                                  chunk_nodes)])
            return carry

        lax.fori_loop(0, n_chunks, chunk_body, 0)

    return body(xt, idx_s, idx_d)


def _conv1x1(xs, xj, We, Wo, bias, NB):
    B, C, N = xs.shape
    O = We.shape[0]
    nblocks = pl.cdiv(N, NB)

    def body(xs_ref, xj_ref, we_ref, wo_ref, b_ref, o_ref):
        acc = lax.dot_general(we_ref[...], xs_ref[0],
                              (((1,), (0,)), ((), ())),
                              preferred_element_type=jnp.float32)
        acc = acc + lax.dot_general(wo_ref[...], xj_ref[0],
                                    (((1,), (1,)), ((), ())),
                                    preferred_element_type=jnp.float32)
        o_ref[0] = jnp.maximum(acc + b_ref[...], 0.0)

    return pl.pallas_call(
        body,
        grid=(B, nblocks),
        in_specs=[
            pl.BlockSpec((1, C, NB), lambda bi, ni: (bi, 0, ni)),
            pl.BlockSpec((1, NB, C), lambda bi, ni: (bi, ni, 0)),
            pl.BlockSpec((O, C), lambda bi, ni: (0, 0)),
            pl.BlockSpec((O, C), lambda bi, ni: (0, 0)),
            pl.BlockSpec((O, 1), lambda bi, ni: (0, 0)),
        ],
        out_specs=pl.BlockSpec((1, O, NB), lambda bi, ni: (bi, 0, ni)),
        out_shape=jax.ShapeDtypeStruct((B, O, N), jnp.float32),
    )(xs, xj, We, Wo, bias)


def kernel(x, edge_index, W, b):
    B, C, N, _ = x.shape
    K = edge_index.shape[-1]
    O = W.shape[0]
    BN = B * N

    npw = BN // NW
    chunk_nodes = 5
    n_chunks = npw // chunk_nodes

    xs = x[..., 0]
    xt = jnp.transpose(xs, (0, 2, 1)).reshape(BN, C)
    offs = (jnp.arange(B, dtype=jnp.int32) * N).reshape(B, 1, 1)
    idx_s = (edge_index[0] + offs).reshape(NW, n_chunks, chunk_nodes * K)
    idx_d = (edge_index[1] + offs).reshape(NW, n_chunks, chunk_nodes * K)

    xj = _gather_max(xt, idx_s, idx_d, n_chunks, chunk_nodes, K, C, npw)
    xj = xj.reshape(B, N, C)

    We = W[:, 0::2]
    Wo = W[:, 1::2]
    out = _conv1x1(xs, xj, We, Wo, b.reshape(O, 1), 2048)
    return out[..., None]
